# baseline (speedup 1.0000x reference)
import functools

import jax
import jax.numpy as jnp
from jax import lax
from jax.experimental import pallas as pl
from jax.experimental.pallas import tpu as pltpu
from jax.experimental.pallas import tpu_sc as plsc

_N = 10000
_NPAD = 10240
_E = 320000
_EPAD = 327680
_F = 128
_H = 256
_HH = _H // 2
_G = 64
_C = 2
_RB = 256

_CH = 128
_EPT16 = _EPAD // 16
_EPT32 = _EPAD // 32
_RPT = _NPAD // 32
_DPT = _NPAD // 16

_MESH = plsc.VectorSubcoreMesh(core_axis_name="c", subcore_axis_name="s")


def _zero_rows(ref, nrows, ncols):
    def body(r, carry):
        for j in range(ncols // 16):
            ref[r, pl.ds(j * 16, 16)] = jnp.zeros((16,), jnp.float32)
        return carry
    lax.fori_loop(0, nrows, body, 0)


def _zero_flat(ref, n):
    def body(r, carry):
        ref[pl.ds(r * 16, 16)] = jnp.zeros((16,), jnp.float32)
        return carry
    lax.fori_loop(0, n // 16, body, 0)


def _sc_stats(dst_pad, batch_pad):
    @functools.partial(
        pl.kernel,
        out_type=[jax.ShapeDtypeStruct((2, _NPAD), jnp.float32),
                  jax.ShapeDtypeStruct((2, 128), jnp.float32)],
        mesh=_MESH,
        scratch_types=[
            pltpu.VMEM((_CH,), jnp.int32),
            pltpu.VMEM((_CH,), jnp.float32),
            pltpu.VMEM((80,), jnp.int32),
            pltpu.VMEM((80,), jnp.float32),
            pltpu.VMEM((_DPT,), jnp.float32),
            pltpu.VMEM_SHARED((_NPAD,), jnp.float32),
            pltpu.VMEM_SHARED((128,), jnp.float32),
        ],
    )
    def k(dst_hbm, batch_hbm, deg_out, cnt_out, idx_v, ones_v, bidx, bones,
          obuf, dacc, cacc):
        c = lax.axis_index("c")
        s = lax.axis_index("s")
        g = c * 16 + s

        for j in range(_CH // 16):
            ones_v[pl.ds(j * 16, 16)] = jnp.ones((16,), jnp.float32)
        for j in range(80 // 16):
            bones[pl.ds(j * 16, 16)] = jnp.ones((16,), jnp.float32)
        _zero_flat(obuf, _DPT)
        pltpu.sync_copy(obuf, dacc.at[pl.ds(s * _DPT, _DPT)])

        @pl.when(s == 0)
        def _():
            pltpu.sync_copy(obuf.at[pl.ds(0, 128)], cacc)

        plsc.subcore_barrier()

        def ebody(i, carry):
            off = g * _EPT32 + i * _CH
            pltpu.sync_copy(dst_hbm.at[pl.ds(off, _CH)], idx_v)
            pltpu.sync_copy(ones_v, dacc.at[idx_v], add=True)
            return carry
        lax.fori_loop(0, _EPT32 // _CH, ebody, 0)

        def bbody(i, carry):
            off = g * _RPT + i * 80
            pltpu.sync_copy(batch_hbm.at[pl.ds(off, 80)], bidx)
            pltpu.sync_copy(bones, cacc.at[bidx], add=True)
            return carry
        lax.fori_loop(0, _RPT // 80, bbody, 0)

        plsc.subcore_barrier()

        pltpu.sync_copy(dacc.at[pl.ds(s * _DPT, _DPT)], obuf)
        pltpu.sync_copy(obuf, deg_out.at[c, pl.ds(s * _DPT, _DPT)])

        @pl.when(s == 0)
        def _():
            pltpu.sync_copy(cacc, ones_v)
            pltpu.sync_copy(ones_v, cnt_out.at[c])

    return k(dst_pad, batch_pad)


_SCH = 64
_SNIT = _EPT16 // _SCH
_SBK = 16
_SNBLK = _SNIT // _SBK
_NBUF = 4


def _sc_spmm(hs_a, hs_b, src3, dst3):
    @functools.partial(
        pl.kernel,
        out_type=[jax.ShapeDtypeStruct((_NPAD, _HH), jnp.float32)] * 2,
        mesh=_MESH,
        scratch_types=[
            pltpu.VMEM((2, _SBK, _SCH), jnp.int32),
            pltpu.VMEM((2, _SBK, _SCH), jnp.int32),
            pltpu.VMEM((_NBUF, _SCH, _HH), jnp.float32),
            pltpu.VMEM_SHARED((_NPAD, _HH), jnp.float32),
            pltpu.SemaphoreType.DMA((_NBUF,)),
            pltpu.SemaphoreType.DMA((_NBUF,)),
        ],
    )
    def k(hsa_hbm, hsb_hbm, src_hbm, dst_hbm, oa_hbm, ob_hbm,
          sblk, dblk, rows, acc, gsem, ssem):
        c = lax.axis_index("c")
        s = lax.axis_index("s")

        def zbody(r, carry):
            for j in range(_HH // 16):
                rows[0, r, pl.ds(j * 16, 16)] = jnp.zeros((16,), jnp.float32)
            return carry
        lax.fori_loop(0, _SCH, zbody, 0)
        for r in range(_DPT // _SCH):
            pltpu.sync_copy(rows.at[0],
                            acc.at[pl.ds(s * _DPT + r * _SCH, _SCH)])
        plsc.subcore_barrier()

        def run(h_hbm, out_hbm):
            pltpu.sync_copy(src_hbm.at[s, 0], sblk.at[0])
            pltpu.sync_copy(dst_hbm.at[s, 0], dblk.at[0])
            for q in range(_NBUF):
                pltpu.async_copy(h_hbm.at[sblk.at[0, q]], rows.at[q],
                                 gsem.at[q])

            def blkbody(blk, carry):
                ib = blk % 2

                @pl.when(blk < _SNBLK - 1)
                def _():
                    pltpu.sync_copy(src_hbm.at[s, blk + 1], sblk.at[1 - ib])
                    pltpu.sync_copy(dst_hbm.at[s, blk + 1], dblk.at[1 - ib])

                for kk in range(_SBK):
                    b = kk % _NBUF
                    j = blk * _SBK + kk
                    pltpu.make_async_copy(h_hbm.at[sblk.at[ib, kk]],
                                          rows.at[b], gsem.at[b]).wait()
                    pltpu.async_copy(rows.at[b], acc.at[dblk.at[ib, kk]],
                                     ssem.at[b], add=True)

                    @pl.when(j < _SNIT - _NBUF)
                    def _():
                        pltpu.make_async_copy(rows.at[b],
                                              acc.at[dblk.at[ib, kk]],
                                              ssem.at[b]).wait()
                        if kk < _SBK - _NBUF:
                            pltpu.async_copy(
                                h_hbm.at[sblk.at[ib, kk + _NBUF]],
                                rows.at[b], gsem.at[b])
                        else:
                            pltpu.async_copy(
                                h_hbm.at[sblk.at[1 - ib, kk + _NBUF - _SBK]],
                                rows.at[b], gsem.at[b])
                return carry
            lax.fori_loop(0, _SNBLK, blkbody, 0)
            for i in range(_NBUF):
                kk = _SBK - _NBUF + i
                pltpu.make_async_copy(rows.at[kk % _NBUF],
                                      acc.at[dblk.at[1, kk]],
                                      ssem.at[kk % _NBUF]).wait()
            plsc.subcore_barrier()
            for r in range(_DPT // _SCH):
                off = s * _DPT + r * _SCH
                pltpu.sync_copy(acc.at[pl.ds(off, _SCH)], rows.at[0])
                pltpu.sync_copy(rows.at[0], out_hbm.at[pl.ds(off, _SCH)])

        @pl.when(c == 0)
        def _():
            run(hsa_hbm, oa_hbm)

        @pl.when(c == 1)
        def _():
            run(hsb_hbm, ob_hbm)

    return k(hs_a, hs_b, src3, dst3)


def _sc_pool(h3_a, h3_b, batch_pad):
    @functools.partial(
        pl.kernel,
        out_type=jax.ShapeDtypeStruct((2, _G, _HH), jnp.float32),
        mesh=_MESH,
        scratch_types=[
            pltpu.VMEM((64,), jnp.int32),
            pltpu.VMEM((64, _HH), jnp.float32),
            pltpu.VMEM_SHARED((128, _HH), jnp.float32),
        ],
    )
    def k(ha_hbm, hb_hbm, batch_hbm, pool_out, bidx, rows, pacc):
        c = lax.axis_index("c")
        s = lax.axis_index("s")

        _zero_rows(rows, 64, _HH)
        pltpu.sync_copy(rows.at[pl.ds(0, 8)], pacc.at[pl.ds(s * 8, 8)])
        plsc.subcore_barrier()

        def run(h_hbm):
            def body(i, carry):
                off = s * (_NPAD // 16) + i * 64
                pltpu.sync_copy(batch_hbm.at[pl.ds(off, 64)], bidx)
                pltpu.sync_copy(h_hbm.at[pl.ds(off, 64)], rows)
                pltpu.sync_copy(rows, pacc.at[bidx], add=True)
                return carry
            lax.fori_loop(0, _NPAD // 16 // 64, body, 0)

        @pl.when(c == 0)
        def _():
            run(ha_hbm)

        @pl.when(c == 1)
        def _():
            run(hb_hbm)

        plsc.subcore_barrier()

        @pl.when(s == 0)
        def _():
            pltpu.sync_copy(pacc.at[pl.ds(0, _G)], rows)
            pltpu.sync_copy(rows, pool_out.at[c])

    return k(h3_a, h3_b, batch_pad)


def _dinv_body(d_ref, o_ref):
    o_ref[...] = lax.rsqrt(1.0 + d_ref[0:1, :] + d_ref[1:2, :])


def _dinv_tc(deg_part):
    return pl.pallas_call(
        _dinv_body,
        in_specs=[pl.BlockSpec((2, _NPAD), lambda: (0, 0))],
        out_specs=pl.BlockSpec((1, _NPAD), lambda: (0, 0)),
        out_shape=jax.ShapeDtypeStruct((1, _NPAD), jnp.float32),
    )(deg_part)


def _scale_mm_body(m_ref, w_ref, dinv_ref, oa_ref, ob_ref):
    t = jnp.dot(m_ref[...], w_ref[...], preferred_element_type=jnp.float32)
    t = t * dinv_ref[...]
    oa_ref[...] = t[:, :_HH]
    ob_ref[...] = t[:, _HH:]


def _scale_mm(M, W, dinv):
    K = M.shape[1]
    return pl.pallas_call(
        _scale_mm_body,
        grid=(_NPAD // _RB,),
        in_specs=[pl.BlockSpec((_RB, K), lambda i: (i, 0)),
                  pl.BlockSpec((K, _H), lambda i: (0, 0)),
                  pl.BlockSpec((_RB, 1), lambda i: (i, 0))],
        out_specs=[pl.BlockSpec((_RB, _HH), lambda i: (i, 0)),
                   pl.BlockSpec((_RB, _HH), lambda i: (i, 0))],
        out_shape=[jax.ShapeDtypeStruct((_NPAD, _HH), jnp.float32)] * 2,
    )(M, W, dinv)


def _layer_body(aa_ref, ab_ref, ha_ref, hb_ref, dinv_ref, b_ref, w_ref,
                oa_ref, ob_ref):
    m = jnp.concatenate([aa_ref[...] + ha_ref[...],
                         ab_ref[...] + hb_ref[...]], axis=1)
    m = jnp.maximum(m * dinv_ref[...] + b_ref[...], 0.0)
    t = jnp.dot(m, w_ref[...], preferred_element_type=jnp.float32)
    t = t * dinv_ref[...]
    oa_ref[...] = t[:, :_HH]
    ob_ref[...] = t[:, _HH:]


def _layer(agg_a, agg_b, hs_a, hs_b, dinv, b, W):
    return pl.pallas_call(
        _layer_body,
        grid=(_NPAD // _RB,),
        in_specs=[pl.BlockSpec((_RB, _HH), lambda i: (i, 0)),
                  pl.BlockSpec((_RB, _HH), lambda i: (i, 0)),
                  pl.BlockSpec((_RB, _HH), lambda i: (i, 0)),
                  pl.BlockSpec((_RB, _HH), lambda i: (i, 0)),
                  pl.BlockSpec((_RB, 1), lambda i: (i, 0)),
                  pl.BlockSpec((1, _H), lambda i: (0, 0)),
                  pl.BlockSpec((_H, _H), lambda i: (0, 0))],
        out_specs=[pl.BlockSpec((_RB, _HH), lambda i: (i, 0)),
                   pl.BlockSpec((_RB, _HH), lambda i: (i, 0))],
        out_shape=[jax.ShapeDtypeStruct((_NPAD, _HH), jnp.float32)] * 2,
    )(agg_a, agg_b, hs_a, hs_b, dinv, b, W)


def _final_body(aa_ref, ab_ref, ha_ref, hb_ref, dinv_ref, b_ref,
                oa_ref, ob_ref):
    m = jnp.concatenate([aa_ref[...] + ha_ref[...],
                         ab_ref[...] + hb_ref[...]], axis=1)
    m = jnp.maximum(m * dinv_ref[...] + b_ref[...], 0.0)
    oa_ref[...] = m[:, :_HH]
    ob_ref[...] = m[:, _HH:]


def _final_layer(agg_a, agg_b, hs_a, hs_b, dinv, b):
    return pl.pallas_call(
        _final_body,
        grid=(_NPAD // _RB,),
        in_specs=[pl.BlockSpec((_RB, _HH), lambda i: (i, 0)),
                  pl.BlockSpec((_RB, _HH), lambda i: (i, 0)),
                  pl.BlockSpec((_RB, _HH), lambda i: (i, 0)),
                  pl.BlockSpec((_RB, _HH), lambda i: (i, 0)),
                  pl.BlockSpec((_RB, 1), lambda i: (i, 0)),
                  pl.BlockSpec((1, _H), lambda i: (0, 0))],
        out_specs=[pl.BlockSpec((_RB, _HH), lambda i: (i, 0)),
                   pl.BlockSpec((_RB, _HH), lambda i: (i, 0))],
        out_shape=[jax.ShapeDtypeStruct((_NPAD, _HH), jnp.float32)] * 2,
    )(agg_a, agg_b, hs_a, hs_b, dinv, b)


def _cls_body(p_ref, c_ref, wc1_ref, bc1_ref, wc2_ref, bc2_ref, o_ref):
    pooled = p_ref[...] / jnp.maximum(c_ref[...], 1.0)
    z = jnp.maximum(
        jnp.dot(pooled, wc1_ref[...], preferred_element_type=jnp.float32)
        + bc1_ref[...], 0.0)
    z2 = jnp.dot(z, wc2_ref[...], preferred_element_type=jnp.float32) \
        + bc2_ref[...]
    col = jax.lax.broadcasted_iota(jnp.int32, (_G, 128), 1)
    z2m = jnp.where(col < _C, z2, -1e30)
    mx = jnp.max(z2m, axis=1, keepdims=True)
    lse = mx + jnp.log(jnp.sum(jnp.exp(z2m - mx), axis=1, keepdims=True))
    o_ref[...] = z2 - lse


def _classifier(pool_sums, counts, Wc1, bc1, Wc2, bc2):
    Wc2p = jnp.pad(Wc2, ((0, 0), (0, 128 - _C)))
    bc2p = jnp.pad(bc2, (0, 128 - _C)).reshape(1, 128)
    return pl.pallas_call(
        _cls_body,
        in_specs=[pl.BlockSpec((_G, _H), lambda: (0, 0)),
                  pl.BlockSpec((_G, 1), lambda: (0, 0)),
                  pl.BlockSpec((_H, _HH), lambda: (0, 0)),
                  pl.BlockSpec((1, _HH), lambda: (0, 0)),
                  pl.BlockSpec((_HH, 128), lambda: (0, 0)),
                  pl.BlockSpec((1, 128), lambda: (0, 0))],
        out_specs=pl.BlockSpec((_G, 128), lambda: (0, 0)),
        out_shape=jax.ShapeDtypeStruct((_G, 128), jnp.float32),
    )(pool_sums, counts, Wc1, bc1.reshape(1, _HH), Wc2p, bc2p)


def kernel(x, edge_index, batch, W1, b1, W2, b2, W3, b3, Wc1, bc1, Wc2, bc2):
    pad_e = _EPAD - _E
    src_pad = jnp.concatenate(
        [edge_index[0], jnp.zeros((pad_e,), jnp.int32)])
    dst_pad = jnp.concatenate(
        [edge_index[1], jnp.full((pad_e,), _N, jnp.int32)])
    batch_pad = jnp.concatenate(
        [batch, jnp.full((_NPAD - _N,), _G, jnp.int32)])

    deg_part, cnt_part = _sc_stats(dst_pad, batch_pad)
    dinv = _dinv_tc(deg_part).reshape(_NPAD, 1)

    xp = jnp.pad(x, ((0, _NPAD - _N), (0, 0)))
    src3 = src_pad.reshape(16, _SNBLK, _SBK, _SCH)
    dst3 = dst_pad.reshape(16, _SNBLK, _SBK, _SCH)

    hs1_a, hs1_b = _scale_mm(xp, W1, dinv)
    ag_a, ag_b = _sc_spmm(hs1_a, hs1_b, src3, dst3)
    hs2_a, hs2_b = _layer(ag_a, ag_b, hs1_a, hs1_b, dinv, b1.reshape(1, _H), W2)
    ag_a, ag_b = _sc_spmm(hs2_a, hs2_b, src3, dst3)
    hs3_a, hs3_b = _layer(ag_a, ag_b, hs2_a, hs2_b, dinv, b2.reshape(1, _H), W3)
    ag_a, ag_b = _sc_spmm(hs3_a, hs3_b, src3, dst3)
    h3_a, h3_b = _final_layer(ag_a, ag_b, hs3_a, hs3_b, dinv, b3.reshape(1, _H))

    pool_part = _sc_pool(h3_a, h3_b, batch_pad)
    sums = jnp.concatenate([pool_part[0], pool_part[1]], axis=1)
    cnts = (cnt_part[0, :_G] + cnt_part[1, :_G]).reshape(_G, 1)

    out = _classifier(sums, cnts, Wc1, bc1, Wc2, bc2)
    return out[:, :_C]


# speedup vs baseline: 1.0143x; 1.0143x over previous
import functools

import jax
import jax.numpy as jnp
from jax import lax
from jax.experimental import pallas as pl
from jax.experimental.pallas import tpu as pltpu
from jax.experimental.pallas import tpu_sc as plsc

_N = 10000
_NPAD = 10240
_E = 320000
_EPAD = 327680
_F = 128
_H = 256
_HH = _H // 2
_G = 64
_C = 2
_RB = 256

_CH = 128
_EPT16 = _EPAD // 16
_EPT32 = _EPAD // 32
_RPT = _NPAD // 32
_DPT = _NPAD // 16

_MESH = plsc.VectorSubcoreMesh(core_axis_name="c", subcore_axis_name="s")


def _zero_rows(ref, nrows, ncols):
    def body(r, carry):
        for j in range(ncols // 16):
            ref[r, pl.ds(j * 16, 16)] = jnp.zeros((16,), jnp.float32)
        return carry
    lax.fori_loop(0, nrows, body, 0)


def _zero_flat(ref, n):
    def body(r, carry):
        ref[pl.ds(r * 16, 16)] = jnp.zeros((16,), jnp.float32)
        return carry
    lax.fori_loop(0, n // 16, body, 0)


def _sc_stats(dst_pad, batch_pad):
    @functools.partial(
        pl.kernel,
        out_type=[jax.ShapeDtypeStruct((2, _NPAD), jnp.float32),
                  jax.ShapeDtypeStruct((2, 128), jnp.float32)],
        mesh=_MESH,
        scratch_types=[
            pltpu.VMEM((_CH,), jnp.int32),
            pltpu.VMEM((_CH,), jnp.float32),
            pltpu.VMEM((80,), jnp.int32),
            pltpu.VMEM((80,), jnp.float32),
            pltpu.VMEM((_DPT,), jnp.float32),
            pltpu.VMEM_SHARED((_NPAD,), jnp.float32),
            pltpu.VMEM_SHARED((128,), jnp.float32),
        ],
    )
    def k(dst_hbm, batch_hbm, deg_out, cnt_out, idx_v, ones_v, bidx, bones,
          obuf, dacc, cacc):
        c = lax.axis_index("c")
        s = lax.axis_index("s")
        g = c * 16 + s

        for j in range(_CH // 16):
            ones_v[pl.ds(j * 16, 16)] = jnp.ones((16,), jnp.float32)
        for j in range(80 // 16):
            bones[pl.ds(j * 16, 16)] = jnp.ones((16,), jnp.float32)
        _zero_flat(obuf, _DPT)
        pltpu.sync_copy(obuf, dacc.at[pl.ds(s * _DPT, _DPT)])

        @pl.when(s == 0)
        def _():
            pltpu.sync_copy(obuf.at[pl.ds(0, 128)], cacc)

        plsc.subcore_barrier()

        def ebody(i, carry):
            off = g * _EPT32 + i * _CH
            pltpu.sync_copy(dst_hbm.at[pl.ds(off, _CH)], idx_v)
            pltpu.sync_copy(ones_v, dacc.at[idx_v], add=True)
            return carry
        lax.fori_loop(0, _EPT32 // _CH, ebody, 0)

        def bbody(i, carry):
            off = g * _RPT + i * 80
            pltpu.sync_copy(batch_hbm.at[pl.ds(off, 80)], bidx)
            pltpu.sync_copy(bones, cacc.at[bidx], add=True)
            return carry
        lax.fori_loop(0, _RPT // 80, bbody, 0)

        plsc.subcore_barrier()

        pltpu.sync_copy(dacc.at[pl.ds(s * _DPT, _DPT)], obuf)
        pltpu.sync_copy(obuf, deg_out.at[c, pl.ds(s * _DPT, _DPT)])

        @pl.when(s == 0)
        def _():
            pltpu.sync_copy(cacc, ones_v)
            pltpu.sync_copy(ones_v, cnt_out.at[c])

    return k(dst_pad, batch_pad)


_SCH = 128
_SNIT = _EPT16 // _SCH
_SBK = 16
_SNBLK = _SNIT // _SBK
_NBUF = 2


def _sc_spmm(hs_a, hs_b, src3, dst3):
    @functools.partial(
        pl.kernel,
        out_type=[jax.ShapeDtypeStruct((_NPAD, _HH), jnp.float32)] * 2,
        mesh=_MESH,
        scratch_types=[
            pltpu.VMEM((2, _SBK, _SCH), jnp.int32),
            pltpu.VMEM((2, _SBK, _SCH), jnp.int32),
            pltpu.VMEM((_NBUF, _SCH, _HH), jnp.float32),
            pltpu.VMEM_SHARED((_NPAD, _HH), jnp.float32),
            pltpu.SemaphoreType.DMA((_NBUF,)),
            pltpu.SemaphoreType.DMA((_NBUF,)),
        ],
    )
    def k(hsa_hbm, hsb_hbm, src_hbm, dst_hbm, oa_hbm, ob_hbm,
          sblk, dblk, rows, acc, gsem, ssem):
        c = lax.axis_index("c")
        s = lax.axis_index("s")

        def zbody(r, carry):
            for j in range(_HH // 16):
                rows[0, r, pl.ds(j * 16, 16)] = jnp.zeros((16,), jnp.float32)
            return carry
        lax.fori_loop(0, _SCH, zbody, 0)
        for r in range(_DPT // _SCH):
            pltpu.sync_copy(rows.at[0],
                            acc.at[pl.ds(s * _DPT + r * _SCH, _SCH)])
        plsc.subcore_barrier()

        def run(h_hbm, out_hbm):
            pltpu.sync_copy(src_hbm.at[s, 0], sblk.at[0])
            pltpu.sync_copy(dst_hbm.at[s, 0], dblk.at[0])
            for q in range(_NBUF):
                pltpu.async_copy(h_hbm.at[sblk.at[0, q]], rows.at[q],
                                 gsem.at[q])

            def blkbody(blk, carry):
                ib = blk % 2

                @pl.when(blk < _SNBLK - 1)
                def _():
                    pltpu.sync_copy(src_hbm.at[s, blk + 1], sblk.at[1 - ib])
                    pltpu.sync_copy(dst_hbm.at[s, blk + 1], dblk.at[1 - ib])

                for kk in range(_SBK):
                    b = kk % _NBUF
                    j = blk * _SBK + kk
                    pltpu.make_async_copy(h_hbm.at[sblk.at[ib, kk]],
                                          rows.at[b], gsem.at[b]).wait()
                    pltpu.async_copy(rows.at[b], acc.at[dblk.at[ib, kk]],
                                     ssem.at[b], add=True)

                    @pl.when(j < _SNIT - _NBUF)
                    def _():
                        pltpu.make_async_copy(rows.at[b],
                                              acc.at[dblk.at[ib, kk]],
                                              ssem.at[b]).wait()
                        if kk < _SBK - _NBUF:
                            pltpu.async_copy(
                                h_hbm.at[sblk.at[ib, kk + _NBUF]],
                                rows.at[b], gsem.at[b])
                        else:
                            pltpu.async_copy(
                                h_hbm.at[sblk.at[1 - ib, kk + _NBUF - _SBK]],
                                rows.at[b], gsem.at[b])
                return carry
            lax.fori_loop(0, _SNBLK, blkbody, 0)
            for i in range(_NBUF):
                kk = _SBK - _NBUF + i
                pltpu.make_async_copy(rows.at[kk % _NBUF],
                                      acc.at[dblk.at[1, kk]],
                                      ssem.at[kk % _NBUF]).wait()
            plsc.subcore_barrier()
            for r in range(_DPT // _SCH):
                off = s * _DPT + r * _SCH
                pltpu.sync_copy(acc.at[pl.ds(off, _SCH)], rows.at[0])
                pltpu.sync_copy(rows.at[0], out_hbm.at[pl.ds(off, _SCH)])

        @pl.when(c == 0)
        def _():
            run(hsa_hbm, oa_hbm)

        @pl.when(c == 1)
        def _():
            run(hsb_hbm, ob_hbm)

    return k(hs_a, hs_b, src3, dst3)


def _sc_pool(h3_a, h3_b, batch_pad):
    @functools.partial(
        pl.kernel,
        out_type=jax.ShapeDtypeStruct((2, _G, _HH), jnp.float32),
        mesh=_MESH,
        scratch_types=[
            pltpu.VMEM((64,), jnp.int32),
            pltpu.VMEM((64, _HH), jnp.float32),
            pltpu.VMEM_SHARED((128, _HH), jnp.float32),
        ],
    )
    def k(ha_hbm, hb_hbm, batch_hbm, pool_out, bidx, rows, pacc):
        c = lax.axis_index("c")
        s = lax.axis_index("s")

        _zero_rows(rows, 64, _HH)
        pltpu.sync_copy(rows.at[pl.ds(0, 8)], pacc.at[pl.ds(s * 8, 8)])
        plsc.subcore_barrier()

        def run(h_hbm):
            def body(i, carry):
                off = s * (_NPAD // 16) + i * 64
                pltpu.sync_copy(batch_hbm.at[pl.ds(off, 64)], bidx)
                pltpu.sync_copy(h_hbm.at[pl.ds(off, 64)], rows)
                pltpu.sync_copy(rows, pacc.at[bidx], add=True)
                return carry
            lax.fori_loop(0, _NPAD // 16 // 64, body, 0)

        @pl.when(c == 0)
        def _():
            run(ha_hbm)

        @pl.when(c == 1)
        def _():
            run(hb_hbm)

        plsc.subcore_barrier()

        @pl.when(s == 0)
        def _():
            pltpu.sync_copy(pacc.at[pl.ds(0, _G)], rows)
            pltpu.sync_copy(rows, pool_out.at[c])

    return k(h3_a, h3_b, batch_pad)


def _dinv_body(d_ref, o_ref):
    o_ref[...] = lax.rsqrt(1.0 + d_ref[0:1, :] + d_ref[1:2, :])


def _dinv_tc(deg_part):
    return pl.pallas_call(
        _dinv_body,
        in_specs=[pl.BlockSpec((2, _NPAD), lambda: (0, 0))],
        out_specs=pl.BlockSpec((1, _NPAD), lambda: (0, 0)),
        out_shape=jax.ShapeDtypeStruct((1, _NPAD), jnp.float32),
    )(deg_part)


def _scale_mm_body(m_ref, w_ref, dinv_ref, oa_ref, ob_ref):
    t = jnp.dot(m_ref[...], w_ref[...], preferred_element_type=jnp.float32)
    t = t * dinv_ref[...]
    oa_ref[...] = t[:, :_HH]
    ob_ref[...] = t[:, _HH:]


def _scale_mm(M, W, dinv):
    K = M.shape[1]
    return pl.pallas_call(
        _scale_mm_body,
        grid=(_NPAD // _RB,),
        in_specs=[pl.BlockSpec((_RB, K), lambda i: (i, 0)),
                  pl.BlockSpec((K, _H), lambda i: (0, 0)),
                  pl.BlockSpec((_RB, 1), lambda i: (i, 0))],
        out_specs=[pl.BlockSpec((_RB, _HH), lambda i: (i, 0)),
                   pl.BlockSpec((_RB, _HH), lambda i: (i, 0))],
        out_shape=[jax.ShapeDtypeStruct((_NPAD, _HH), jnp.float32)] * 2,
    )(M, W, dinv)


def _layer_body(aa_ref, ab_ref, ha_ref, hb_ref, dinv_ref, b_ref, w_ref,
                oa_ref, ob_ref):
    m = jnp.concatenate([aa_ref[...] + ha_ref[...],
                         ab_ref[...] + hb_ref[...]], axis=1)
    m = jnp.maximum(m * dinv_ref[...] + b_ref[...], 0.0)
    t = jnp.dot(m, w_ref[...], preferred_element_type=jnp.float32)
    t = t * dinv_ref[...]
    oa_ref[...] = t[:, :_HH]
    ob_ref[...] = t[:, _HH:]


def _layer(agg_a, agg_b, hs_a, hs_b, dinv, b, W):
    return pl.pallas_call(
        _layer_body,
        grid=(_NPAD // _RB,),
        in_specs=[pl.BlockSpec((_RB, _HH), lambda i: (i, 0)),
                  pl.BlockSpec((_RB, _HH), lambda i: (i, 0)),
                  pl.BlockSpec((_RB, _HH), lambda i: (i, 0)),
                  pl.BlockSpec((_RB, _HH), lambda i: (i, 0)),
                  pl.BlockSpec((_RB, 1), lambda i: (i, 0)),
                  pl.BlockSpec((1, _H), lambda i: (0, 0)),
                  pl.BlockSpec((_H, _H), lambda i: (0, 0))],
        out_specs=[pl.BlockSpec((_RB, _HH), lambda i: (i, 0)),
                   pl.BlockSpec((_RB, _HH), lambda i: (i, 0))],
        out_shape=[jax.ShapeDtypeStruct((_NPAD, _HH), jnp.float32)] * 2,
    )(agg_a, agg_b, hs_a, hs_b, dinv, b, W)


def _final_body(aa_ref, ab_ref, ha_ref, hb_ref, dinv_ref, b_ref,
                oa_ref, ob_ref):
    m = jnp.concatenate([aa_ref[...] + ha_ref[...],
                         ab_ref[...] + hb_ref[...]], axis=1)
    m = jnp.maximum(m * dinv_ref[...] + b_ref[...], 0.0)
    oa_ref[...] = m[:, :_HH]
    ob_ref[...] = m[:, _HH:]


def _final_layer(agg_a, agg_b, hs_a, hs_b, dinv, b):
    return pl.pallas_call(
        _final_body,
        grid=(_NPAD // _RB,),
        in_specs=[pl.BlockSpec((_RB, _HH), lambda i: (i, 0)),
                  pl.BlockSpec((_RB, _HH), lambda i: (i, 0)),
                  pl.BlockSpec((_RB, _HH), lambda i: (i, 0)),
                  pl.BlockSpec((_RB, _HH), lambda i: (i, 0)),
                  pl.BlockSpec((_RB, 1), lambda i: (i, 0)),
                  pl.BlockSpec((1, _H), lambda i: (0, 0))],
        out_specs=[pl.BlockSpec((_RB, _HH), lambda i: (i, 0)),
                   pl.BlockSpec((_RB, _HH), lambda i: (i, 0))],
        out_shape=[jax.ShapeDtypeStruct((_NPAD, _HH), jnp.float32)] * 2,
    )(agg_a, agg_b, hs_a, hs_b, dinv, b)


def _cls_body(p_ref, c_ref, wc1_ref, bc1_ref, wc2_ref, bc2_ref, o_ref):
    pooled = p_ref[...] / jnp.maximum(c_ref[...], 1.0)
    z = jnp.maximum(
        jnp.dot(pooled, wc1_ref[...], preferred_element_type=jnp.float32)
        + bc1_ref[...], 0.0)
    z2 = jnp.dot(z, wc2_ref[...], preferred_element_type=jnp.float32) \
        + bc2_ref[...]
    col = jax.lax.broadcasted_iota(jnp.int32, (_G, 128), 1)
    z2m = jnp.where(col < _C, z2, -1e30)
    mx = jnp.max(z2m, axis=1, keepdims=True)
    lse = mx + jnp.log(jnp.sum(jnp.exp(z2m - mx), axis=1, keepdims=True))
    o_ref[...] = z2 - lse


def _classifier(pool_sums, counts, Wc1, bc1, Wc2, bc2):
    Wc2p = jnp.pad(Wc2, ((0, 0), (0, 128 - _C)))
    bc2p = jnp.pad(bc2, (0, 128 - _C)).reshape(1, 128)
    return pl.pallas_call(
        _cls_body,
        in_specs=[pl.BlockSpec((_G, _H), lambda: (0, 0)),
                  pl.BlockSpec((_G, 1), lambda: (0, 0)),
                  pl.BlockSpec((_H, _HH), lambda: (0, 0)),
                  pl.BlockSpec((1, _HH), lambda: (0, 0)),
                  pl.BlockSpec((_HH, 128), lambda: (0, 0)),
                  pl.BlockSpec((1, 128), lambda: (0, 0))],
        out_specs=pl.BlockSpec((_G, 128), lambda: (0, 0)),
        out_shape=jax.ShapeDtypeStruct((_G, 128), jnp.float32),
    )(pool_sums, counts, Wc1, bc1.reshape(1, _HH), Wc2p, bc2p)


def kernel(x, edge_index, batch, W1, b1, W2, b2, W3, b3, Wc1, bc1, Wc2, bc2):
    pad_e = _EPAD - _E
    src_pad = jnp.concatenate(
        [edge_index[0], jnp.zeros((pad_e,), jnp.int32)])
    dst_pad = jnp.concatenate(
        [edge_index[1], jnp.full((pad_e,), _N, jnp.int32)])
    batch_pad = jnp.concatenate(
        [batch, jnp.full((_NPAD - _N,), _G, jnp.int32)])

    deg_part, cnt_part = _sc_stats(dst_pad, batch_pad)
    dinv = _dinv_tc(deg_part).reshape(_NPAD, 1)

    xp = jnp.pad(x, ((0, _NPAD - _N), (0, 0)))
    src3 = src_pad.reshape(16, _SNBLK, _SBK, _SCH)
    dst3 = dst_pad.reshape(16, _SNBLK, _SBK, _SCH)

    hs1_a, hs1_b = _scale_mm(xp, W1, dinv)
    ag_a, ag_b = _sc_spmm(hs1_a, hs1_b, src3, dst3)
    hs2_a, hs2_b = _layer(ag_a, ag_b, hs1_a, hs1_b, dinv, b1.reshape(1, _H), W2)
    ag_a, ag_b = _sc_spmm(hs2_a, hs2_b, src3, dst3)
    hs3_a, hs3_b = _layer(ag_a, ag_b, hs2_a, hs2_b, dinv, b2.reshape(1, _H), W3)
    ag_a, ag_b = _sc_spmm(hs3_a, hs3_b, src3, dst3)
    h3_a, h3_b = _final_layer(ag_a, ag_b, hs3_a, hs3_b, dinv, b3.reshape(1, _H))

    pool_part = _sc_pool(h3_a, h3_b, batch_pad)
    sums = jnp.concatenate([pool_part[0], pool_part[1]], axis=1)
    cnts = (cnt_part[0, :_G] + cnt_part[1, :_G]).reshape(_G, 1)

    out = _classifier(sums, cnts, Wc1, bc1, Wc2, bc2)
    return out[:, :_C]


# speedup vs baseline: 1.0160x; 1.0017x over previous
import functools

import jax
import jax.numpy as jnp
from jax import lax
from jax.experimental import pallas as pl
from jax.experimental.pallas import tpu as pltpu
from jax.experimental.pallas import tpu_sc as plsc

_N = 10000
_NPAD = 10240
_E = 320000
_EPAD = 327680
_F = 128
_H = 256
_HH = _H // 2
_G = 64
_C = 2
_RB = 256

_CH = 128
_EPT16 = _EPAD // 16
_EPT32 = _EPAD // 32
_RPT = _NPAD // 32
_DPT = _NPAD // 16

_MESH = plsc.VectorSubcoreMesh(core_axis_name="c", subcore_axis_name="s")


def _zero_rows(ref, nrows, ncols):
    def body(r, carry):
        for j in range(ncols // 16):
            ref[r, pl.ds(j * 16, 16)] = jnp.zeros((16,), jnp.float32)
        return carry
    lax.fori_loop(0, nrows, body, 0)


def _zero_flat(ref, n):
    def body(r, carry):
        ref[pl.ds(r * 16, 16)] = jnp.zeros((16,), jnp.float32)
        return carry
    lax.fori_loop(0, n // 16, body, 0)


def _sc_stats(dst_pad, batch_pad):
    @functools.partial(
        pl.kernel,
        out_type=[jax.ShapeDtypeStruct((2, _NPAD), jnp.float32),
                  jax.ShapeDtypeStruct((2, 128), jnp.float32)],
        mesh=_MESH,
        scratch_types=[
            pltpu.VMEM((_CH,), jnp.int32),
            pltpu.VMEM((_CH,), jnp.float32),
            pltpu.VMEM((80,), jnp.int32),
            pltpu.VMEM((80,), jnp.float32),
            pltpu.VMEM((_DPT,), jnp.float32),
            pltpu.VMEM_SHARED((_NPAD,), jnp.float32),
            pltpu.VMEM_SHARED((128,), jnp.float32),
        ],
    )
    def k(dst_hbm, batch_hbm, deg_out, cnt_out, idx_v, ones_v, bidx, bones,
          obuf, dacc, cacc):
        c = lax.axis_index("c")
        s = lax.axis_index("s")
        g = c * 16 + s

        for j in range(_CH // 16):
            ones_v[pl.ds(j * 16, 16)] = jnp.ones((16,), jnp.float32)
        for j in range(80 // 16):
            bones[pl.ds(j * 16, 16)] = jnp.ones((16,), jnp.float32)
        _zero_flat(obuf, _DPT)
        pltpu.sync_copy(obuf, dacc.at[pl.ds(s * _DPT, _DPT)])

        @pl.when(s == 0)
        def _():
            pltpu.sync_copy(obuf.at[pl.ds(0, 128)], cacc)

        plsc.subcore_barrier()

        def ebody(i, carry):
            off = g * _EPT32 + i * _CH
            pltpu.sync_copy(dst_hbm.at[pl.ds(off, _CH)], idx_v)
            pltpu.sync_copy(ones_v, dacc.at[idx_v], add=True)
            return carry
        lax.fori_loop(0, _EPT32 // _CH, ebody, 0)

        def bbody(i, carry):
            off = g * _RPT + i * 80
            pltpu.sync_copy(batch_hbm.at[pl.ds(off, 80)], bidx)
            pltpu.sync_copy(bones, cacc.at[bidx], add=True)
            return carry
        lax.fori_loop(0, _RPT // 80, bbody, 0)

        plsc.subcore_barrier()

        pltpu.sync_copy(dacc.at[pl.ds(s * _DPT, _DPT)], obuf)
        pltpu.sync_copy(obuf, deg_out.at[c, pl.ds(s * _DPT, _DPT)])

        @pl.when(s == 0)
        def _():
            pltpu.sync_copy(cacc, ones_v)
            pltpu.sync_copy(ones_v, cnt_out.at[c])

    return k(dst_pad, batch_pad)


_SCH = 128
_SNIT = _EPT16 // _SCH
_SBK = 16
_SNBLK = _SNIT // _SBK
_NBUF = 2


def _sc_spmm(hs_a, hs_b, src3, dst3):
    @functools.partial(
        pl.kernel,
        out_type=[jax.ShapeDtypeStruct((_NPAD, _HH), jnp.float32)] * 2,
        mesh=_MESH,
        scratch_types=[
            pltpu.VMEM((2, _SBK, _SCH), jnp.int32),
            pltpu.VMEM((2, _SBK, _SCH), jnp.int32),
            pltpu.VMEM((_NBUF, _SCH, _HH), jnp.float32),
            pltpu.VMEM_SHARED((_NPAD, _HH), jnp.float32),
            pltpu.SemaphoreType.DMA((_NBUF,)),
            pltpu.SemaphoreType.DMA((_NBUF,)),
        ],
    )
    def k(hsa_hbm, hsb_hbm, src_hbm, dst_hbm, oa_hbm, ob_hbm,
          sblk, dblk, rows, acc, gsem, ssem):
        c = lax.axis_index("c")
        s = lax.axis_index("s")

        def zbody(r, carry):
            for j in range(_HH // 16):
                rows[0, r, pl.ds(j * 16, 16)] = jnp.zeros((16,), jnp.float32)
            return carry
        lax.fori_loop(0, _SCH, zbody, 0)
        for r in range(_DPT // _SCH):
            pltpu.sync_copy(rows.at[0],
                            acc.at[pl.ds(s * _DPT + r * _SCH, _SCH)])
        plsc.subcore_barrier()

        def run(h_hbm, out_hbm):
            pltpu.sync_copy(src_hbm.at[s, 0], sblk.at[0])
            pltpu.sync_copy(dst_hbm.at[s, 0], dblk.at[0])
            for q in range(_NBUF):
                pltpu.async_copy(h_hbm.at[sblk.at[0, q]], rows.at[q],
                                 gsem.at[q])

            def blkbody(blk, carry):
                ib = blk % 2

                @pl.when(blk < _SNBLK - 1)
                def _():
                    pltpu.sync_copy(src_hbm.at[s, blk + 1], sblk.at[1 - ib])
                    pltpu.sync_copy(dst_hbm.at[s, blk + 1], dblk.at[1 - ib])

                for kk in range(_SBK):
                    b = kk % _NBUF
                    j = blk * _SBK + kk
                    pltpu.make_async_copy(h_hbm.at[sblk.at[ib, kk]],
                                          rows.at[b], gsem.at[b]).wait()
                    pltpu.async_copy(rows.at[b], acc.at[dblk.at[ib, kk]],
                                     ssem.at[b], add=True)

                    @pl.when(j < _SNIT - _NBUF)
                    def _():
                        pltpu.make_async_copy(rows.at[b],
                                              acc.at[dblk.at[ib, kk]],
                                              ssem.at[b]).wait()
                        if kk < _SBK - _NBUF:
                            pltpu.async_copy(
                                h_hbm.at[sblk.at[ib, kk + _NBUF]],
                                rows.at[b], gsem.at[b])
                        else:
                            pltpu.async_copy(
                                h_hbm.at[sblk.at[1 - ib, kk + _NBUF - _SBK]],
                                rows.at[b], gsem.at[b])
                return carry
            lax.fori_loop(0, _SNBLK, blkbody, 0)
            for i in range(_NBUF):
                kk = _SBK - _NBUF + i
                pltpu.make_async_copy(rows.at[kk % _NBUF],
                                      acc.at[dblk.at[1, kk]],
                                      ssem.at[kk % _NBUF]).wait()
            plsc.subcore_barrier()
            pltpu.sync_copy(acc.at[pl.ds(s * _DPT, _DPT)],
                            out_hbm.at[pl.ds(s * _DPT, _DPT)])

        @pl.when(c == 0)
        def _():
            run(hsa_hbm, oa_hbm)

        @pl.when(c == 1)
        def _():
            run(hsb_hbm, ob_hbm)

    return k(hs_a, hs_b, src3, dst3)


def _sc_pool(h3_a, h3_b, batch_pad):
    @functools.partial(
        pl.kernel,
        out_type=jax.ShapeDtypeStruct((2, _G, _HH), jnp.float32),
        mesh=_MESH,
        scratch_types=[
            pltpu.VMEM((64,), jnp.int32),
            pltpu.VMEM((64, _HH), jnp.float32),
            pltpu.VMEM_SHARED((128, _HH), jnp.float32),
        ],
    )
    def k(ha_hbm, hb_hbm, batch_hbm, pool_out, bidx, rows, pacc):
        c = lax.axis_index("c")
        s = lax.axis_index("s")

        _zero_rows(rows, 64, _HH)
        pltpu.sync_copy(rows.at[pl.ds(0, 8)], pacc.at[pl.ds(s * 8, 8)])
        plsc.subcore_barrier()

        def run(h_hbm):
            def body(i, carry):
                off = s * (_NPAD // 16) + i * 64
                pltpu.sync_copy(batch_hbm.at[pl.ds(off, 64)], bidx)
                pltpu.sync_copy(h_hbm.at[pl.ds(off, 64)], rows)
                pltpu.sync_copy(rows, pacc.at[bidx], add=True)
                return carry
            lax.fori_loop(0, _NPAD // 16 // 64, body, 0)

        @pl.when(c == 0)
        def _():
            run(ha_hbm)

        @pl.when(c == 1)
        def _():
            run(hb_hbm)

        plsc.subcore_barrier()

        @pl.when(s == 0)
        def _():
            pltpu.sync_copy(pacc.at[pl.ds(0, _G)], rows)
            pltpu.sync_copy(rows, pool_out.at[c])

    return k(h3_a, h3_b, batch_pad)


def _dinv_body(d_ref, o_ref):
    o_ref[...] = lax.rsqrt(1.0 + d_ref[0:1, :] + d_ref[1:2, :])


def _dinv_tc(deg_part):
    return pl.pallas_call(
        _dinv_body,
        in_specs=[pl.BlockSpec((2, _NPAD), lambda: (0, 0))],
        out_specs=pl.BlockSpec((1, _NPAD), lambda: (0, 0)),
        out_shape=jax.ShapeDtypeStruct((1, _NPAD), jnp.float32),
    )(deg_part)


def _scale_mm_body(m_ref, w_ref, dinv_ref, oa_ref, ob_ref):
    t = jnp.dot(m_ref[...], w_ref[...], preferred_element_type=jnp.float32)
    t = t * dinv_ref[...]
    oa_ref[...] = t[:, :_HH]
    ob_ref[...] = t[:, _HH:]


def _scale_mm(M, W, dinv):
    K = M.shape[1]
    return pl.pallas_call(
        _scale_mm_body,
        grid=(_NPAD // _RB,),
        in_specs=[pl.BlockSpec((_RB, K), lambda i: (i, 0)),
                  pl.BlockSpec((K, _H), lambda i: (0, 0)),
                  pl.BlockSpec((_RB, 1), lambda i: (i, 0))],
        out_specs=[pl.BlockSpec((_RB, _HH), lambda i: (i, 0)),
                   pl.BlockSpec((_RB, _HH), lambda i: (i, 0))],
        out_shape=[jax.ShapeDtypeStruct((_NPAD, _HH), jnp.float32)] * 2,
    )(M, W, dinv)


def _layer_body(aa_ref, ab_ref, ha_ref, hb_ref, dinv_ref, b_ref, w_ref,
                oa_ref, ob_ref):
    m = jnp.concatenate([aa_ref[...] + ha_ref[...],
                         ab_ref[...] + hb_ref[...]], axis=1)
    m = jnp.maximum(m * dinv_ref[...] + b_ref[...], 0.0)
    t = jnp.dot(m, w_ref[...], preferred_element_type=jnp.float32)
    t = t * dinv_ref[...]
    oa_ref[...] = t[:, :_HH]
    ob_ref[...] = t[:, _HH:]


def _layer(agg_a, agg_b, hs_a, hs_b, dinv, b, W):
    return pl.pallas_call(
        _layer_body,
        grid=(_NPAD // _RB,),
        in_specs=[pl.BlockSpec((_RB, _HH), lambda i: (i, 0)),
                  pl.BlockSpec((_RB, _HH), lambda i: (i, 0)),
                  pl.BlockSpec((_RB, _HH), lambda i: (i, 0)),
                  pl.BlockSpec((_RB, _HH), lambda i: (i, 0)),
                  pl.BlockSpec((_RB, 1), lambda i: (i, 0)),
                  pl.BlockSpec((1, _H), lambda i: (0, 0)),
                  pl.BlockSpec((_H, _H), lambda i: (0, 0))],
        out_specs=[pl.BlockSpec((_RB, _HH), lambda i: (i, 0)),
                   pl.BlockSpec((_RB, _HH), lambda i: (i, 0))],
        out_shape=[jax.ShapeDtypeStruct((_NPAD, _HH), jnp.float32)] * 2,
    )(agg_a, agg_b, hs_a, hs_b, dinv, b, W)


def _final_body(aa_ref, ab_ref, ha_ref, hb_ref, dinv_ref, b_ref,
                oa_ref, ob_ref):
    m = jnp.concatenate([aa_ref[...] + ha_ref[...],
                         ab_ref[...] + hb_ref[...]], axis=1)
    m = jnp.maximum(m * dinv_ref[...] + b_ref[...], 0.0)
    oa_ref[...] = m[:, :_HH]
    ob_ref[...] = m[:, _HH:]


def _final_layer(agg_a, agg_b, hs_a, hs_b, dinv, b):
    return pl.pallas_call(
        _final_body,
        grid=(_NPAD // _RB,),
        in_specs=[pl.BlockSpec((_RB, _HH), lambda i: (i, 0)),
                  pl.BlockSpec((_RB, _HH), lambda i: (i, 0)),
                  pl.BlockSpec((_RB, _HH), lambda i: (i, 0)),
                  pl.BlockSpec((_RB, _HH), lambda i: (i, 0)),
                  pl.BlockSpec((_RB, 1), lambda i: (i, 0)),
                  pl.BlockSpec((1, _H), lambda i: (0, 0))],
        out_specs=[pl.BlockSpec((_RB, _HH), lambda i: (i, 0)),
                   pl.BlockSpec((_RB, _HH), lambda i: (i, 0))],
        out_shape=[jax.ShapeDtypeStruct((_NPAD, _HH), jnp.float32)] * 2,
    )(agg_a, agg_b, hs_a, hs_b, dinv, b)


def _cls_body(p_ref, c_ref, wc1_ref, bc1_ref, wc2_ref, bc2_ref, o_ref):
    pooled = p_ref[...] / jnp.maximum(c_ref[...], 1.0)
    z = jnp.maximum(
        jnp.dot(pooled, wc1_ref[...], preferred_element_type=jnp.float32)
        + bc1_ref[...], 0.0)
    z2 = jnp.dot(z, wc2_ref[...], preferred_element_type=jnp.float32) \
        + bc2_ref[...]
    col = jax.lax.broadcasted_iota(jnp.int32, (_G, 128), 1)
    z2m = jnp.where(col < _C, z2, -1e30)
    mx = jnp.max(z2m, axis=1, keepdims=True)
    lse = mx + jnp.log(jnp.sum(jnp.exp(z2m - mx), axis=1, keepdims=True))
    o_ref[...] = z2 - lse


def _classifier(pool_sums, counts, Wc1, bc1, Wc2, bc2):
    Wc2p = jnp.pad(Wc2, ((0, 0), (0, 128 - _C)))
    bc2p = jnp.pad(bc2, (0, 128 - _C)).reshape(1, 128)
    return pl.pallas_call(
        _cls_body,
        in_specs=[pl.BlockSpec((_G, _H), lambda: (0, 0)),
                  pl.BlockSpec((_G, 1), lambda: (0, 0)),
                  pl.BlockSpec((_H, _HH), lambda: (0, 0)),
                  pl.BlockSpec((1, _HH), lambda: (0, 0)),
                  pl.BlockSpec((_HH, 128), lambda: (0, 0)),
                  pl.BlockSpec((1, 128), lambda: (0, 0))],
        out_specs=pl.BlockSpec((_G, 128), lambda: (0, 0)),
        out_shape=jax.ShapeDtypeStruct((_G, 128), jnp.float32),
    )(pool_sums, counts, Wc1, bc1.reshape(1, _HH), Wc2p, bc2p)


def kernel(x, edge_index, batch, W1, b1, W2, b2, W3, b3, Wc1, bc1, Wc2, bc2):
    pad_e = _EPAD - _E
    src_pad = jnp.concatenate(
        [edge_index[0], jnp.zeros((pad_e,), jnp.int32)])
    dst_pad = jnp.concatenate(
        [edge_index[1], jnp.full((pad_e,), _N, jnp.int32)])
    batch_pad = jnp.concatenate(
        [batch, jnp.full((_NPAD - _N,), _G, jnp.int32)])

    deg_part, cnt_part = _sc_stats(dst_pad, batch_pad)
    dinv = _dinv_tc(deg_part).reshape(_NPAD, 1)

    xp = jnp.pad(x, ((0, _NPAD - _N), (0, 0)))
    src3 = src_pad.reshape(16, _SNBLK, _SBK, _SCH)
    dst3 = dst_pad.reshape(16, _SNBLK, _SBK, _SCH)

    hs1_a, hs1_b = _scale_mm(xp, W1, dinv)
    ag_a, ag_b = _sc_spmm(hs1_a, hs1_b, src3, dst3)
    hs2_a, hs2_b = _layer(ag_a, ag_b, hs1_a, hs1_b, dinv, b1.reshape(1, _H), W2)
    ag_a, ag_b = _sc_spmm(hs2_a, hs2_b, src3, dst3)
    hs3_a, hs3_b = _layer(ag_a, ag_b, hs2_a, hs2_b, dinv, b2.reshape(1, _H), W3)
    ag_a, ag_b = _sc_spmm(hs3_a, hs3_b, src3, dst3)
    h3_a, h3_b = _final_layer(ag_a, ag_b, hs3_a, hs3_b, dinv, b3.reshape(1, _H))

    pool_part = _sc_pool(h3_a, h3_b, batch_pad)
    sums = jnp.concatenate([pool_part[0], pool_part[1]], axis=1)
    cnts = (cnt_part[0, :_G] + cnt_part[1, :_G]).reshape(_G, 1)

    out = _classifier(sums, cnts, Wc1, bc1, Wc2, bc2)
    return out[:, :_C]
